# baseline (speedup 1.0000x reference)
import functools

import jax
import jax.numpy as jnp
from jax import lax
from jax.experimental import pallas as pl
from jax.experimental.pallas import tpu as pltpu
from jax.experimental.pallas import tpu_sc as plsc

BATCH = 16384
EMB = 32
EPS = 1e-5

NC = 2
NS = 16
NW = NC * NS
BPW = BATCH // NW
CB = 64
NCHUNK = BPW // CB
QW = 4 * EMB


def _extract_rows(blocks_v, soff_v, rows_v, so_base, col):
    iota = lax.iota(jnp.int32, 16)
    for j0 in range(0, CB, 16):
        j_vec = iota + j0
        c_base = soff_v[pl.ds(so_base + j0, 16)] * EMB
        for d in range(EMB):
            vals = plsc.load_gather(blocks_v, [j_vec, c_base + d])
            plsc.store_scatter(rows_v, [j_vec, iota * 0 + (col * EMB + d)],
                               vals)


def _sc_gather_body(ublk, usub, iblk, isub, Ug, Ig, Um, Im, out,
                    blk_u, blk_i, sub_u, sub_i, blocks_v, rows_v, sem):
    wid = lax.axis_index("s") * NC + lax.axis_index("c")
    base = wid * BPW
    pltpu.sync_copy(ublk.at[pl.ds(base, BPW)], blk_u)
    pltpu.sync_copy(usub.at[pl.ds(base, BPW)], sub_u)
    pltpu.sync_copy(iblk.at[pl.ds(base, BPW)], blk_i)
    pltpu.sync_copy(isub.at[pl.ds(base, BPW)], sub_i)

    tbls = ((Ug, blk_u, sub_u, 0), (Ig, blk_i, sub_i, 1),
            (Um, blk_u, sub_u, 2), (Im, blk_i, sub_i, 3))

    def chunk(c, _):
        off = c * CB
        for tbl, blk_v, sub_v, col in tbls:
            pltpu.async_copy(tbl.at[blk_v.at[pl.ds(off, CB)]], blocks_v,
                             sem).wait()
            _extract_rows(blocks_v, sub_v, rows_v, off, col)
        pltpu.sync_copy(rows_v, out.at[pl.ds(base + off, CB)])
        return ()

    lax.fori_loop(0, NCHUNK, chunk, (), unroll=False)


def _sc_gather(ublk, usub, iblk, isub, Ug, Ig, Um, Im):
    mesh = plsc.VectorSubcoreMesh(core_axis_name="c", subcore_axis_name="s",
                                  num_cores=NC, num_subcores=NS)
    f = pl.kernel(
        _sc_gather_body,
        out_type=jax.ShapeDtypeStruct((BATCH, QW), jnp.float32),
        mesh=mesh,
        compiler_params=pltpu.CompilerParams(needs_layout_passes=False),
        scratch_types=[
            pltpu.VMEM((BPW,), jnp.int32),
            pltpu.VMEM((BPW,), jnp.int32),
            pltpu.VMEM((BPW,), jnp.int32),
            pltpu.VMEM((BPW,), jnp.int32),
            pltpu.VMEM((CB, QW), jnp.float32),
            pltpu.VMEM((CB, QW), jnp.float32),
            pltpu.SemaphoreType.DMA,
        ],
    )
    return f(ublk, usub, iblk, isub, Ug.reshape(-1, QW), Ig.reshape(-1, QW),
             Um.reshape(-1, QW), Im.reshape(-1, QW))


def _tc_tail_body(g_ref, w1_ref, b1_ref, g1_ref, be1_ref, w2_ref, b2_ref,
                  g2_ref, be2_ref, wh_ref, bh_ref, out_ref):
    f32 = jnp.float32
    ug = g_ref[:, 0 * EMB:1 * EMB]
    ig = g_ref[:, 1 * EMB:2 * EMB]
    um = g_ref[:, 2 * EMB:3 * EMB]
    im = g_ref[:, 3 * EMB:4 * EMB]
    w1 = w1_ref[...]
    inv1 = g1_ref[...] / jnp.sqrt(1.0 + EPS)
    inv2 = g2_ref[...] / jnp.sqrt(1.0 + EPS)
    h = lax.dot_general(um, w1[:, :EMB], (((1,), (1,)), ((), ())),
                        preferred_element_type=f32)
    h += lax.dot_general(im, w1[:, EMB:], (((1,), (1,)), ((), ())),
                         preferred_element_type=f32)
    h = (h + b1_ref[...]) * inv1 + be1_ref[...]
    h = jnp.maximum(h, 0.0)
    h = lax.dot_general(h, w2_ref[...], (((1,), (1,)), ((), ())),
                        preferred_element_type=f32)
    h = (h + b2_ref[...]) * inv2 + be2_ref[...]
    h = jnp.maximum(h, 0.0)
    wh = wh_ref[...]
    gmf = ug * ig
    out = jnp.sum(gmf * wh[:, :EMB], axis=1) + jnp.sum(h * wh[:, EMB:], axis=1)
    out = out + bh_ref[0, 0]
    out_ref[...] = jnp.clip(out, -2.0, 2.0)


def _tc_tail(g, W1, b1, g1, be1, W2, b2, g2, be2, Wh, bh):
    blk = 2048
    grid = (BATCH // blk,)
    rows = pl.BlockSpec((blk, QW), lambda i: (i, 0))
    full = lambda a: pl.BlockSpec(a.shape, lambda i: (0,) * a.ndim)
    args = (W1, b1, g1, be1, W2, b2, g2, be2, Wh, bh)
    return pl.pallas_call(
        _tc_tail_body,
        grid=grid,
        in_specs=[rows] + [full(a) for a in args],
        out_specs=pl.BlockSpec((blk,), lambda i: (i,)),
        out_shape=jax.ShapeDtypeStruct((BATCH,), jnp.float32),
    )(g, *args)


def kernel(x, Ug, Ig, Um, Im, W1, b1, g1, be1, W2, b2, g2, be2, Wh, bh):
    xi = x.astype(jnp.int32)
    uidx = xi[:, 0]
    iidx = xi[:, 1]
    g = _sc_gather(uidx // 4, uidx % 4, iidx // 4, iidx % 4,
                   Ug, Ig, Um, Im)
    return _tc_tail(g,
                    W1, b1.reshape(1, -1), g1.reshape(1, -1),
                    be1.reshape(1, -1), W2, b2.reshape(1, -1),
                    g2.reshape(1, -1), be2.reshape(1, -1), Wh,
                    bh.reshape(1, -1))


# speedup vs baseline: 1.4763x; 1.4763x over previous
import functools

import jax
import jax.numpy as jnp
from jax import lax
from jax.experimental import pallas as pl
from jax.experimental.pallas import tpu as pltpu
from jax.experimental.pallas import tpu_sc as plsc

BATCH = 16384
EMB = 32
EPS = 1e-5

NC = 2
NS = 16
NW = NC * NS
BPW = BATCH // NW
CHUNK = 128
NCHUNK = BPW // CHUNK


def _sc_gather_body(uidx, iidx, Ug, Ig, Um, Im,
                    o_ug, o_ig, o_um, o_im,
                    vu, vi, st_ug, st_ig, st_um, st_im, sem):
    wid = lax.axis_index("s") * NC + lax.axis_index("c")
    base = wid * BPW
    pltpu.sync_copy(uidx.at[pl.ds(base, BPW)], vu)
    pltpu.sync_copy(iidx.at[pl.ds(base, BPW)], vi)

    def chunk(c, _):
        off = c * CHUNK

        @plsc.parallel_loop(0, CHUNK // 16, step=1, unroll=2)
        def group(g):
            o2 = off + g * 16
            uvec = vu[pl.ds(o2, 16)]
            ivec = vi[pl.ds(o2, 16)]
            for k in range(16):
                u = uvec[k]
                t = ivec[k]
                r = g * 16 + k
                pltpu.make_async_copy(Ug.at[u], st_ug.at[r], sem).start()
                pltpu.make_async_copy(Um.at[u], st_um.at[r], sem).start()
                pltpu.make_async_copy(Ig.at[t], st_ig.at[r], sem).start()
                pltpu.make_async_copy(Im.at[t], st_im.at[r], sem).start()
        for st in (st_ug, st_um, st_ig, st_im):
            pltpu.make_async_copy(Ug.at[pl.ds(0, CHUNK)], st, sem).wait()
        dst = pl.ds(base + off, CHUNK)
        pltpu.sync_copy(st_ug, o_ug.at[dst])
        pltpu.sync_copy(st_um, o_um.at[dst])
        pltpu.sync_copy(st_ig, o_ig.at[dst])
        pltpu.sync_copy(st_im, o_im.at[dst])
        return ()

    lax.fori_loop(0, NCHUNK, chunk, (), unroll=False)


def _sc_gather(uidx, iidx, Ug, Ig, Um, Im):
    mesh = plsc.VectorSubcoreMesh(core_axis_name="c", subcore_axis_name="s",
                                  num_cores=NC, num_subcores=NS)
    row = jax.ShapeDtypeStruct((BATCH, EMB), jnp.float32)
    f = pl.kernel(
        _sc_gather_body,
        out_type=(row, row, row, row),
        mesh=mesh,
        compiler_params=pltpu.CompilerParams(use_tc_tiling_on_sc=True),
        scratch_types=[
            pltpu.VMEM((BPW,), jnp.int32),
            pltpu.VMEM((BPW,), jnp.int32),
            pltpu.VMEM((CHUNK, EMB), jnp.float32),
            pltpu.VMEM((CHUNK, EMB), jnp.float32),
            pltpu.VMEM((CHUNK, EMB), jnp.float32),
            pltpu.VMEM((CHUNK, EMB), jnp.float32),
            pltpu.SemaphoreType.DMA,
        ],
    )
    return f(uidx, iidx, Ug, Ig, Um, Im)


def _tc_tail_body(ug_ref, ig_ref, um_ref, im_ref, w1_ref, b1_ref, g1_ref,
                  be1_ref, w2_ref, b2_ref, g2_ref, be2_ref, wh_ref, bh_ref,
                  out_ref):
    f32 = jnp.float32
    um = um_ref[...]
    im = im_ref[...]
    w1 = w1_ref[...]
    inv1 = g1_ref[...] / jnp.sqrt(1.0 + EPS)
    inv2 = g2_ref[...] / jnp.sqrt(1.0 + EPS)
    h = lax.dot_general(um, w1[:, :EMB], (((1,), (1,)), ((), ())),
                        preferred_element_type=f32)
    h += lax.dot_general(im, w1[:, EMB:], (((1,), (1,)), ((), ())),
                         preferred_element_type=f32)
    h = (h + b1_ref[...]) * inv1 + be1_ref[...]
    h = jnp.maximum(h, 0.0)
    h = lax.dot_general(h, w2_ref[...], (((1,), (1,)), ((), ())),
                        preferred_element_type=f32)
    h = (h + b2_ref[...]) * inv2 + be2_ref[...]
    h = jnp.maximum(h, 0.0)
    wh = wh_ref[...]
    gmf = ug_ref[...] * ig_ref[...]
    out = jnp.sum(gmf * wh[:, :EMB], axis=1) + jnp.sum(h * wh[:, EMB:], axis=1)
    out = out + bh_ref[0, 0]
    out_ref[...] = jnp.clip(out, -2.0, 2.0)


def _tc_tail(ug, ig, um, im, W1, b1, g1, be1, W2, b2, g2, be2, Wh, bh):
    blk = 2048
    grid = (BATCH // blk,)
    rows = pl.BlockSpec((blk, EMB), lambda i: (i, 0))
    full = lambda a: pl.BlockSpec(a.shape, lambda i: (0,) * a.ndim)
    args = (W1, b1, g1, be1, W2, b2, g2, be2, Wh, bh)
    return pl.pallas_call(
        _tc_tail_body,
        grid=grid,
        in_specs=[rows, rows, rows, rows] + [full(a) for a in args],
        out_specs=pl.BlockSpec((blk,), lambda i: (i,)),
        out_shape=jax.ShapeDtypeStruct((BATCH,), jnp.float32),
    )(ug, ig, um, im, *args)


def kernel(x, Ug, Ig, Um, Im, W1, b1, g1, be1, W2, b2, g2, be2, Wh, bh):
    xi = x.astype(jnp.int32)
    uidx = xi[:, 0]
    iidx = xi[:, 1]
    ug, ig, um, im = _sc_gather(uidx, iidx, Ug, Ig, Um, Im)
    return _tc_tail(ug, ig, um, im,
                    W1, b1.reshape(1, -1), g1.reshape(1, -1),
                    be1.reshape(1, -1), W2, b2.reshape(1, -1),
                    g2.reshape(1, -1), be2.reshape(1, -1), Wh,
                    bh.reshape(1, -1))
